# baseline (speedup 1.0000x reference)
import functools

import jax
import jax.numpy as jnp
from jax import lax
from jax.experimental import pallas as pl
from jax.experimental.pallas import tpu as pltpu
from jax.experimental.pallas import tpu_sc as plsc

N = 10000
E = 320000
D = 128
R = 16
B = 4

NC = 2
NS = 16
NW = NC * NS
C = 128
NBUF = 3
NCH = 81
EPW = NCH * C
E_PAD = EPW * NW
STRIPE = 632
NBLK = 1000
NB = N // NBLK


def _table_body(comb_ref, wb_ref, x_ref, o_ref):
    r_id = pl.program_id(0)
    w = (comb_ref[r_id, 0] * wb_ref[0]
         + comb_ref[r_id, 1] * wb_ref[1]
         + comb_ref[r_id, 2] * wb_ref[2]
         + comb_ref[r_id, 3] * wb_ref[3])
    o_ref[0] = jnp.dot(x_ref[...], w.astype(jnp.bfloat16),
                       preferred_element_type=jnp.float32)


def _make_table(comb, wb, x_bf):
    return pl.pallas_call(
        _table_body,
        grid=(R, NB),
        in_specs=[
            pl.BlockSpec(memory_space=pltpu.SMEM),
            pl.BlockSpec((B, D, D), lambda r, i: (0, 0, 0)),
            pl.BlockSpec((NBLK, D), lambda r, i: (i, 0)),
        ],
        out_specs=pl.BlockSpec((1, NBLK, D), lambda r, i: (r, i, 0)),
        out_shape=jax.ShapeDtypeStruct((R, N, D), jnp.float32),
    )(comb, wb, x_bf)


def _combine_body(a0_ref, a1_ref, x_ref, wl_ref, b_ref, o_ref, obf_ref):
    acc = (a0_ref[...] + a1_ref[...]
           + jnp.dot(x_ref[...], wl_ref[...], preferred_element_type=jnp.float32)
           + b_ref[...])
    out = jnp.maximum(acc, 0.0)
    o_ref[...] = out
    obf_ref[...] = out.astype(jnp.bfloat16)


def _combine(a0, a1, x, wl, bias2d):
    return pl.pallas_call(
        _combine_body,
        grid=(NB,),
        in_specs=[
            pl.BlockSpec((NBLK, D), lambda i: (i, 0)),
            pl.BlockSpec((NBLK, D), lambda i: (i, 0)),
            pl.BlockSpec((NBLK, D), lambda i: (i, 0)),
            pl.BlockSpec((D, D), lambda i: (0, 0)),
            pl.BlockSpec((1, D), lambda i: (0, 0)),
        ],
        out_specs=[pl.BlockSpec((NBLK, D), lambda i: (i, 0)),
                   pl.BlockSpec((NBLK, D), lambda i: (i, 0))],
        out_shape=[jax.ShapeDtypeStruct((N, D), jnp.float32),
                   jax.ShapeDtypeStruct((N, D), jnp.bfloat16)],
    )(a0, a1, x, wl, bias2d)


_SPLAT_DN = lax.GatherDimensionNumbers(
    offset_dims=(), collapsed_slice_dims=(0,), start_index_map=(0,))


def _splat16(vec, l):
    idx = jnp.full((16, 1), l, jnp.int32)
    return lax.gather(vec, idx, _SPLAT_DN, (1,),
                      mode=lax.GatherScatterMode.PROMISE_IN_BOUNDS)


def _sc_body(table_hbm, epack_hbm, out0_hbm, out1_hbm,
             ep_db, rows0, rows1, rows2,
             agg, se0, se1, se2, sg0, sg1, sg2, ss0, ss1, ss2, sz):
    c = lax.axis_index("c")
    s = lax.axis_index("s")
    wid = c * NS + s
    rows = (rows0, rows1, rows2)
    se = (se0, se1, se2)
    sg = (sg0, sg1, sg2)
    ss = (ss0, ss1, ss2)

    def _fetch_pack(ch, b):
        pltpu.async_copy(epack_hbm.at[wid, ch], ep_db.at[b], se[b])

    def _wait_pack(ch, b):
        pltpu.make_async_copy(epack_hbm.at[wid, ch], ep_db.at[b],
                              se[b]).wait()

    def _issue_gather(ch, b):
        pltpu.async_copy(table_hbm.at[ep_db.at[b, 2]], rows[b], sg[b])

    def _wait_gather(ch, b):
        pltpu.make_async_copy(table_hbm.at[ep_db.at[b, 2]], rows[b],
                              sg[b]).wait()

    def _wait_scatter(b):
        pltpu.make_async_copy(rows[b], agg.at[ep_db.at[b, 0]], ss[b]).wait()

    _fetch_pack(0, 0)
    _fetch_pack(1, 1)
    _wait_pack(0, 0)
    _issue_gather(0, 0)

    def _zrow(i, carry):
        for j in range(8):
            rows2[i, pl.ds(j * 16, 16)] = jnp.zeros((16,), jnp.float32)
        return carry
    lax.fori_loop(0, C, _zrow, 0)
    base = pl.multiple_of(s * 625 - (s * 625) % 8, 8)
    for k in range(4):
        pltpu.async_copy(rows2, agg.at[pl.ds(base + k * C, C)], sz)
    pltpu.async_copy(rows2.at[pl.ds(0, STRIPE - 4 * C)],
                     agg.at[pl.ds(base + 4 * C, STRIPE - 4 * C)], sz)
    for k in range(4):
        pltpu.make_async_copy(rows2, agg.at[pl.ds(base + k * C, C)],
                              sz).wait()
    pltpu.make_async_copy(rows2.at[pl.ds(0, STRIPE - 4 * C)],
                          agg.at[pl.ds(base + 4 * C, STRIPE - 4 * C)],
                          sz).wait()

    plsc.subcore_barrier()

    @pl.loop(0, NCH, step=NBUF)
    def _pipeline(ch0):
        for b in range(NBUF):
            ch = ch0 + b
            _wait_gather(ch, b)

            for g in range(C // 16):
                nvec = lax.bitcast_convert_type(
                    ep_db[b, 1, pl.ds(g * 16, 16)], jnp.float32)
                for l in range(16):
                    nb = _splat16(nvec, l)
                    i = g * 16 + l
                    for j in range(8):
                        sl = pl.ds(j * 16, 16)
                        rows[b][i, sl] = rows[b][i, sl] * nb
            pltpu.async_copy(rows[b], agg.at[ep_db.at[b, 0]], ss[b],
                             add=True)

            b2 = (b + 2) % NBUF
            b1 = (b + 1) % NBUF

            @pl.when(ch + 2 < NCH)
            def _prefetch_next():
                @pl.when(ch >= 1)
                def _drain_prev():
                    _wait_scatter(b2)
                _fetch_pack(ch + 2, b2)

            @pl.when(ch + 1 < NCH)
            def _gather_next():
                _wait_pack(ch + 1, b1)
                _issue_gather(ch + 1, b1)

    for b in range(NBUF):
        _wait_scatter(b)

    plsc.subcore_barrier()

    @pl.when(c == 0)
    def _dump0():
        pltpu.sync_copy(agg.at[pl.ds(base, STRIPE)],
                        out0_hbm.at[pl.ds(base, STRIPE)])

    @pl.when(c == 1)
    def _dump1():
        pltpu.sync_copy(agg.at[pl.ds(base, STRIPE)],
                        out1_hbm.at[pl.ds(base, STRIPE)])


_sc_call = functools.partial(
    pl.kernel,
    out_type=[jax.ShapeDtypeStruct((N, D), jnp.float32),
              jax.ShapeDtypeStruct((N, D), jnp.float32)],
    mesh=plsc.VectorSubcoreMesh(core_axis_name="c", subcore_axis_name="s",
                                num_cores=NC, num_subcores=NS),
    scratch_types=[
        pltpu.VMEM((NBUF, 3, C), jnp.int32),
        pltpu.VMEM((C, D), jnp.float32),
        pltpu.VMEM((C, D), jnp.float32),
        pltpu.VMEM((C, D), jnp.float32),
        pltpu.VMEM_SHARED((N, D), jnp.float32),
    ] + [pltpu.SemaphoreType.DMA] * 10,
)(_sc_body)


def _layer(x, x_bf, wb, comb, wl, bias, epack):
    table = _make_table(comb, wb, x_bf)
    p0, p1 = _sc_call(table.reshape(R * N, D), epack)
    return _combine(p0, p1, x, wl, bias.reshape(1, D))


def kernel(h, edge_index, r, norm,
           W_basis0, comb0, W_loop0, bias0,
           W_basis1, comb1, W_loop1, bias1):
    src = edge_index[0]
    dst = edge_index[1]
    pad = E_PAD - E
    spread = jnp.arange(pad, dtype=jnp.int32)
    gidx_w = jnp.concatenate(
        [r * N + src, spread % (R * N)]).reshape(NW, NCH, C)
    dst_w = jnp.concatenate([dst, spread % N]).reshape(NW, NCH, C)
    norm_w = lax.bitcast_convert_type(
        jnp.pad(norm[:, 0], (0, pad)), jnp.int32).reshape(NW, NCH, C)
    epack = jnp.stack([dst_w, norm_w, gidx_w], axis=2)

    h1, h1_bf = _layer(h, h.astype(jnp.bfloat16), W_basis0, comb0,
                       W_loop0, bias0, epack)
    h2, _ = _layer(h1, h1_bf, W_basis1, comb1, W_loop1, bias1, epack)
    return h2


# speedup vs baseline: 1.2564x; 1.2564x over previous
import functools

import jax
import jax.numpy as jnp
from jax import lax
from jax.experimental import pallas as pl
from jax.experimental.pallas import tpu as pltpu
from jax.experimental.pallas import tpu_sc as plsc

N = 10000
E = 320000
D = 128
R = 16
B = 4

NC = 2
NS = 16
NW = NC * NS
C = 128
NBUF = 3
NCH = 81
EPW = NCH * C
E_PAD = EPW * NW
STRIPE = 632
NBLK = 1000
NB = N // NBLK


def _table_body(comb_ref, wb_ref, x_ref, o_ref):
    r_id = pl.program_id(0)
    w = (comb_ref[r_id, 0] * wb_ref[0]
         + comb_ref[r_id, 1] * wb_ref[1]
         + comb_ref[r_id, 2] * wb_ref[2]
         + comb_ref[r_id, 3] * wb_ref[3])
    o_ref[0] = jnp.dot(x_ref[...], w.astype(jnp.bfloat16),
                       preferred_element_type=jnp.float32)


def _make_table(comb, wb, x_bf):
    return pl.pallas_call(
        _table_body,
        grid=(R, NB),
        in_specs=[
            pl.BlockSpec(memory_space=pltpu.SMEM),
            pl.BlockSpec((B, D, D), lambda r, i: (0, 0, 0)),
            pl.BlockSpec((NBLK, D), lambda r, i: (i, 0)),
        ],
        out_specs=pl.BlockSpec((1, NBLK, D), lambda r, i: (r, i, 0)),
        out_shape=jax.ShapeDtypeStruct((R, N, D), jnp.float32),
    )(comb, wb, x_bf)


def _combine_body(a0_ref, a1_ref, x_ref, wl_ref, b_ref, o_ref, obf_ref):
    acc = (a0_ref[...] + a1_ref[...]
           + jnp.dot(x_ref[...], wl_ref[...], preferred_element_type=jnp.float32)
           + b_ref[...])
    out = jnp.maximum(acc, 0.0)
    o_ref[...] = out
    obf_ref[...] = out.astype(jnp.bfloat16)


def _combine(a0, a1, x, wl, bias2d):
    return pl.pallas_call(
        _combine_body,
        grid=(NB,),
        in_specs=[
            pl.BlockSpec((NBLK, D), lambda i: (i, 0)),
            pl.BlockSpec((NBLK, D), lambda i: (i, 0)),
            pl.BlockSpec((NBLK, D), lambda i: (i, 0)),
            pl.BlockSpec((D, D), lambda i: (0, 0)),
            pl.BlockSpec((1, D), lambda i: (0, 0)),
        ],
        out_specs=[pl.BlockSpec((NBLK, D), lambda i: (i, 0)),
                   pl.BlockSpec((NBLK, D), lambda i: (i, 0))],
        out_shape=[jax.ShapeDtypeStruct((N, D), jnp.float32),
                   jax.ShapeDtypeStruct((N, D), jnp.bfloat16)],
    )(a0, a1, x, wl, bias2d)


_SPLAT_DN = lax.GatherDimensionNumbers(
    offset_dims=(), collapsed_slice_dims=(0,), start_index_map=(0,))


def _splat16(vec, l):
    idx = jnp.full((16, 1), l, jnp.int32)
    return lax.gather(vec, idx, _SPLAT_DN, (1,),
                      mode=lax.GatherScatterMode.PROMISE_IN_BOUNDS)


def _sc_body(table_hbm, gidx_hbm, dst_hbm, norm_hbm, out0_hbm, out1_hbm,
             gidx_q, dst_db, norm_db, rows0, rows1, rows2,
             agg, se0, se1, se2, sg0, sg1, sg2, ss0, ss1, ss2,
             sq0, sq1, sq2, sz):
    c = lax.axis_index("c")
    s = lax.axis_index("s")
    wid = c * NS + s
    rows = (rows0, rows1, rows2)
    se = (se0, se1, se2)
    sg = (sg0, sg1, sg2)
    ss = (ss0, ss1, ss2)
    sq = (sq0, sq1, sq2)

    def _fetch_gidx(ch, b):
        pltpu.async_copy(gidx_hbm.at[wid, pl.ds(ch * C, C)], gidx_q.at[b],
                         sq[b])

    def _wait_gidx(ch, b):
        pltpu.make_async_copy(gidx_hbm.at[wid, pl.ds(ch * C, C)],
                              gidx_q.at[b], sq[b]).wait()

    def _issue(ch, b):
        pltpu.async_copy(dst_hbm.at[wid, ch], dst_db.at[b], se[b])
        pltpu.async_copy(norm_hbm.at[wid, pl.ds(ch * C, C)], norm_db.at[b],
                         se[b])
        pltpu.async_copy(table_hbm.at[gidx_q.at[b]], rows[b], sg[b])

    def _wait_chunk(ch, b):
        pltpu.make_async_copy(dst_hbm.at[wid, ch], dst_db.at[b],
                              se[b]).wait()
        pltpu.make_async_copy(norm_hbm.at[wid, pl.ds(ch * C, C)],
                              norm_db.at[b], se[b]).wait()
        pltpu.make_async_copy(table_hbm.at[gidx_q.at[b]], rows[b],
                              sg[b]).wait()

    def _wait_scatter(b):
        pltpu.make_async_copy(rows[b], agg.at[dst_db.at[b]], ss[b]).wait()

    for b in range(NBUF):
        _fetch_gidx(b, b)
    _wait_gidx(0, 0)
    _issue(0, 0)
    _wait_gidx(1, 1)
    _issue(1, 1)

    def _zrow(i, carry):
        for j in range(8):
            rows2[i, pl.ds(j * 16, 16)] = jnp.zeros((16,), jnp.float32)
        return carry
    lax.fori_loop(0, C, _zrow, 0)
    base = pl.multiple_of(s * 625 - (s * 625) % 8, 8)
    for k in range(4):
        pltpu.async_copy(rows2, agg.at[pl.ds(base + k * C, C)], sz)
    pltpu.async_copy(rows2.at[pl.ds(0, STRIPE - 4 * C)],
                     agg.at[pl.ds(base + 4 * C, STRIPE - 4 * C)], sz)
    for k in range(4):
        pltpu.make_async_copy(rows2, agg.at[pl.ds(base + k * C, C)],
                              sz).wait()
    pltpu.make_async_copy(rows2.at[pl.ds(0, STRIPE - 4 * C)],
                          agg.at[pl.ds(base + 4 * C, STRIPE - 4 * C)],
                          sz).wait()

    plsc.subcore_barrier()

    @pl.loop(0, NCH, step=NBUF)
    def _pipeline(ch0):
        for b in range(NBUF):
            ch = ch0 + b
            _wait_chunk(ch, b)

            @pl.when(ch + NBUF < NCH)
            def _refill_gidx():
                _fetch_gidx(ch + NBUF, b)

            for g in range(C // 16):
                nvec = norm_db[b, pl.ds(g * 16, 16)]
                for l in range(16):
                    nb = _splat16(nvec, l)
                    i = g * 16 + l
                    for j in range(8):
                        sl = pl.ds(j * 16, 16)
                        rows[b][i, sl] = rows[b][i, sl] * nb
            pltpu.async_copy(rows[b], agg.at[dst_db.at[b]], ss[b], add=True)

            b2 = (b + 2) % NBUF

            @pl.when(ch + 2 < NCH)
            def _prefetch_next():
                @pl.when(ch >= 1)
                def _drain_prev():
                    _wait_scatter(b2)
                _wait_gidx(ch + 2, b2)
                _issue(ch + 2, b2)

    for b in range(NBUF):
        _wait_scatter(b)

    plsc.subcore_barrier()

    @pl.when(c == 0)
    def _dump0():
        pltpu.sync_copy(agg.at[pl.ds(base, STRIPE)],
                        out0_hbm.at[pl.ds(base, STRIPE)])

    @pl.when(c == 1)
    def _dump1():
        pltpu.sync_copy(agg.at[pl.ds(base, STRIPE)],
                        out1_hbm.at[pl.ds(base, STRIPE)])


_sc_call = functools.partial(
    pl.kernel,
    out_type=[jax.ShapeDtypeStruct((N, D), jnp.float32),
              jax.ShapeDtypeStruct((N, D), jnp.float32)],
    mesh=plsc.VectorSubcoreMesh(core_axis_name="c", subcore_axis_name="s",
                                num_cores=NC, num_subcores=NS),
    scratch_types=[
        pltpu.VMEM((NBUF, C), jnp.int32),
        pltpu.VMEM((NBUF, C), jnp.int32),
        pltpu.VMEM((NBUF, C), jnp.float32),
        pltpu.VMEM((C, D), jnp.float32),
        pltpu.VMEM((C, D), jnp.float32),
        pltpu.VMEM((C, D), jnp.float32),
        pltpu.VMEM_SHARED((N, D), jnp.float32),
    ] + [pltpu.SemaphoreType.DMA] * 13,
)(_sc_body)


def _layer(x, x_bf, wb, comb, wl, bias, gidx_w, dst_w, norm_w):
    table = _make_table(comb, wb, x_bf)
    p0, p1 = _sc_call(table.reshape(R * N, D), gidx_w, dst_w, norm_w)
    return _combine(p0, p1, x, wl, bias.reshape(1, D))


def kernel(h, edge_index, r, norm,
           W_basis0, comb0, W_loop0, bias0,
           W_basis1, comb1, W_loop1, bias1):
    src = edge_index[0]
    dst = edge_index[1]
    pad = E_PAD - E
    spread = jnp.arange(pad, dtype=jnp.int32)
    gidx_w = jnp.concatenate(
        [r * N + src, spread % (R * N)]).reshape(NW, EPW)
    dst_w = jnp.concatenate([dst, spread % N]).reshape(NW, NCH, C)
    norm_w = jnp.pad(norm[:, 0], (0, pad)).reshape(NW, EPW)

    h1, h1_bf = _layer(h, h.astype(jnp.bfloat16), W_basis0, comb0,
                       W_loop0, bias0, gidx_w, dst_w, norm_w)
    h2, _ = _layer(h1, h1_bf, W_basis1, comb1, W_loop1, bias1,
                   gidx_w, dst_w, norm_w)
    return h2
